# baseline (speedup 1.0000x reference)
import functools

import jax
import jax.numpy as jnp
from jax import lax
from jax.experimental import pallas as pl
from jax.experimental.pallas import tpu as pltpu
from jax.experimental.pallas import tpu_sc as plsc

B, L = 1024, 200
NPOS = B * L
K = 16
NITEM = 100001
NM, NWK = 24, 52
MSTRIDE = NM * K
WSTRIDE = NWK * K
MROWS = NITEM * MSTRIDE // 128
WROWS = NITEM * WSTRIDE // 128
WTAIL = WROWS * 128
NW = 32
POS_PER_W = NPOS // NW
CHUNK = 400
NCHUNK = POS_PER_W // CHUNK
GROUPS = CHUNK // 16


def _sc_gather(items, t1, t2, m128, w128, sliver):
    mesh = plsc.VectorSubcoreMesh(core_axis_name="c", subcore_axis_name="s")

    @functools.partial(
        pl.kernel,
        out_type=jax.ShapeDtypeStruct((NPOS * 2 * K,), jnp.float32),
        mesh=mesh,
        compiler_params=pltpu.CompilerParams(needs_layout_passes=False),
        scratch_types=[
            pltpu.VMEM((CHUNK,), jnp.int32),
            pltpu.VMEM((CHUNK,), jnp.int32),
            pltpu.VMEM((CHUNK,), jnp.int32),
            pltpu.VMEM((CHUNK,), jnp.int32),
            pltpu.VMEM((CHUNK,), jnp.int32),
            pltpu.VMEM((CHUNK, 128), jnp.float32),
            pltpu.VMEM((CHUNK, 128), jnp.float32),
            pltpu.VMEM((CHUNK * 2 * K,), jnp.float32),
            pltpu.VMEM((64,), jnp.float32),
            pltpu.SemaphoreType.DMA,
            pltpu.SemaphoreType.DMA,
        ],
    )
    def body(items_h, t1_h, t2_h, m_h, w_h, sliver_h, o_h,
             it_v, t1_v, t2_v, im_v, iw_v, gm, gw, ob, sl_v, sem_m, sem_w):
        wid = lax.axis_index("s") * 2 + lax.axis_index("c")
        lane = lax.iota(jnp.int32, 16)
        pltpu.sync_copy(sliver_h, sl_v)

        @pl.loop(0, NCHUNK)
        def _chunk(c):
            base = wid * POS_PER_W + c * CHUNK
            pltpu.sync_copy(items_h.at[pl.ds(base, CHUNK)], it_v)
            pltpu.sync_copy(t1_h.at[pl.ds(base, CHUNK)], t1_v)
            pltpu.sync_copy(t2_h.at[pl.ds(base, CHUNK)], t2_v)

            @pl.loop(0, GROUPS)
            def _idx(g):
                it = it_v[pl.ds(g * 16, 16)]
                om = it * MSTRIDE + t1_v[pl.ds(g * 16, 16)] * K
                ow = it * WSTRIDE + t2_v[pl.ds(g * 16, 16)] * K
                im_v[pl.ds(g * 16, 16)] = lax.shift_right_logical(om, 7)
                iw_v[pl.ds(g * 16, 16)] = jnp.minimum(
                    lax.shift_right_logical(ow, 7), WROWS - 1)

            cm = pltpu.async_copy(m_h.at[im_v], gm, sem_m)
            cw = pltpu.async_copy(w_h.at[iw_v], gw, sem_w)
            cm.wait()
            cw.wait()

            @pl.loop(0, GROUPS)
            def _extract(g):
                rowv = g * 16 + lane
                it = it_v[pl.ds(g * 16, 16)]
                om = it * MSTRIDE + t1_v[pl.ds(g * 16, 16)] * K
                ow = it * WSTRIDE + t2_v[pl.ds(g * 16, 16)] * K
                colm = om & 127
                colw = ow & 127
                tail = ow >= WTAIL
                soff = jnp.maximum(ow - WTAIL, 0)
                tgt = rowv * (2 * K)
                for k in range(K):
                    vm = plsc.load_gather(gm, [rowv, colm + k])
                    plsc.store_scatter(ob, [tgt + k], vm)
                    vw = plsc.load_gather(gw, [rowv, colw + k])
                    vf = plsc.load_gather(sl_v, [soff + k])
                    plsc.store_scatter(
                        ob, [tgt + (K + k)], jnp.where(tail, vf, vw))

            pltpu.sync_copy(ob, o_h.at[pl.ds(base * 2 * K, CHUNK * 2 * K)])

    return body(items, t1, t2, m128, w128, sliver)


def kernel(log_seqs, time1_seqs, time2_seqs, month_pop_table, week_pop_table):
    items = log_seqs.reshape(-1).astype(jnp.int32)
    t1 = time1_seqs.reshape(-1).astype(jnp.int32)
    t2 = time2_seqs.reshape(-1).astype(jnp.int32)
    m128 = month_pop_table.T.reshape(MROWS, 128)
    w128 = week_pop_table.T.reshape(-1)[:WTAIL].reshape(WROWS, 128)
    sliver = week_pop_table[WSTRIDE - 64:, NITEM - 1]
    flat = _sc_gather(items, t1, t2, m128, w128, sliver)
    return flat.reshape(B, L, 2 * K)


# speedup vs baseline: 1.6978x; 1.6978x over previous
import functools

import jax
import jax.numpy as jnp
from jax import lax
from jax.experimental import pallas as pl
from jax.experimental.pallas import tpu as pltpu
from jax.experimental.pallas import tpu_sc as plsc

B, L = 1024, 200
NPOS = B * L
K = 16
NITEM = 100001
NM, NWK = 24, 52
MROWS = 3
WAROWS = 6
WSPLIT = WAROWS * 8
NW = 32
POS_PER_W = NPOS // NW
CHUNK = 256
NCHUNK = POS_PER_W // CHUNK
GROUPS = CHUNK // 16


def _sc_gather(items, t1, t2, m128, wa, wb):
    mesh = plsc.VectorSubcoreMesh(core_axis_name="c", subcore_axis_name="s")

    @functools.partial(
        pl.kernel,
        out_type=jax.ShapeDtypeStruct((NPOS * 2 * K,), jnp.float32),
        mesh=mesh,
        compiler_params=pltpu.CompilerParams(needs_layout_passes=False),
        scratch_types=[
            pltpu.VMEM((CHUNK,), jnp.int32),
            pltpu.VMEM((CHUNK,), jnp.int32),
            pltpu.VMEM((CHUNK,), jnp.int32),
            pltpu.VMEM((CHUNK,), jnp.int32),
            pltpu.VMEM((CHUNK,), jnp.int32),
            pltpu.VMEM((CHUNK,), jnp.int32),
            pltpu.VMEM((CHUNK, 128), jnp.float32),
            pltpu.VMEM((CHUNK, 128), jnp.float32),
            pltpu.VMEM((CHUNK, 128), jnp.float32),
            pltpu.VMEM((CHUNK * 2 * K,), jnp.float32),
            pltpu.SemaphoreType.DMA,
            pltpu.SemaphoreType.DMA,
            pltpu.SemaphoreType.DMA,
        ],
    )
    def body(items_h, t1_h, t2_h, m_h, wa_h, wb_h, o_h,
             it_v, t1_v, t2_v, im_v, ia_v, ib_v, gm, ga, gb, ob,
             sem_m, sem_a, sem_b):
        wid = lax.axis_index("s") * 2 + lax.axis_index("c")
        lane = lax.iota(jnp.int32, 16)

        @pl.loop(0, NCHUNK)
        def _chunk(c):
            base = wid * POS_PER_W + c * CHUNK
            pltpu.sync_copy(items_h.at[pl.ds(base, CHUNK)], it_v)
            pltpu.sync_copy(t1_h.at[pl.ds(base, CHUNK)], t1_v)
            pltpu.sync_copy(t2_h.at[pl.ds(base, CHUNK)], t2_v)

            @pl.loop(0, GROUPS)
            def _idx(g):
                it = it_v[pl.ds(g * 16, 16)]
                im_v[pl.ds(g * 16, 16)] = (
                    it * MROWS + lax.shift_right_logical(t1_v[pl.ds(g * 16, 16)], 3))
                ia_v[pl.ds(g * 16, 16)] = it * WAROWS + jnp.minimum(
                    lax.shift_right_logical(t2_v[pl.ds(g * 16, 16)], 3), WAROWS - 1)
                ib_v[pl.ds(g * 16, 16)] = it

            cm = pltpu.async_copy(m_h.at[im_v], gm, sem_m)
            ca = pltpu.async_copy(wa_h.at[ia_v], ga, sem_a)
            cb = pltpu.async_copy(wb_h.at[ib_v], gb, sem_b)
            cm.wait()
            ca.wait()
            cb.wait()

            @pl.loop(0, GROUPS)
            def _extract(g):
                rowv = g * 16 + lane
                t2g = t2_v[pl.ds(g * 16, 16)]
                colm = (t1_v[pl.ds(g * 16, 16)] & 7) * 16
                cola = (t2g & 7) * 16
                colb = jnp.maximum(t2g - WSPLIT, 0) * 16
                tail = t2g >= WSPLIT
                tgt = rowv * (2 * K)
                for k in range(K):
                    vm = plsc.load_gather(gm, [rowv, colm + k])
                    plsc.store_scatter(ob, [tgt + k], vm)
                    va = plsc.load_gather(ga, [rowv, cola + k])
                    vb = plsc.load_gather(gb, [rowv, colb + k])
                    plsc.store_scatter(
                        ob, [tgt + (K + k)], jnp.where(tail, vb, va))

            pltpu.sync_copy(ob, o_h.at[pl.ds(base * 2 * K, CHUNK * 2 * K)])

    return body(items, t1, t2, m128, wa, wb)


def kernel(log_seqs, time1_seqs, time2_seqs, month_pop_table, week_pop_table):
    items = log_seqs.reshape(-1).astype(jnp.int32)
    t1 = time1_seqs.reshape(-1).astype(jnp.int32)
    t2 = time2_seqs.reshape(-1).astype(jnp.int32)
    m128 = month_pop_table.T.reshape(NITEM * MROWS, 128)
    wa = week_pop_table.T[:, :WAROWS * 128].reshape(NITEM * WAROWS, 128)
    wb = jnp.pad(week_pop_table[WAROWS * 128:, :].T, ((0, 0), (0, 64)))
    flat = _sc_gather(items, t1, t2, m128, wa, wb)
    return flat.reshape(B, L, 2 * K)
